# Initial kernel scaffold; baseline (speedup 1.0000x reference)
import jax
import jax.numpy as jnp
from jax.experimental import pallas as pl


def kernel(hidden, target, W, b, path_nodes, path_bits, path_mask):
    raise NotImplementedError("write your pallas kernel here")


# speedup vs baseline: 1.8885x; 1.8885x over previous
import functools

import jax
import jax.numpy as jnp
from jax import lax
from jax.experimental import pallas as pl
from jax.experimental.pallas import tpu as pltpu
from jax.experimental.pallas import tpu_sc as plsc


def _dlogit_kernel(h_ref, w_ref, bt_ref, out_ref):
    wd = w_ref[1] - w_ref[0]
    bd = bt_ref[1] - bt_ref[0]
    acc = lax.dot_general(
        h_ref[...], wd, (((1,), (0,)), ((), ())),
        preferred_element_type=jnp.float32,
    )
    out_ref[...] = acc + bd[None, :]


def _dlogit_matmul(h, wt, bt):
    N, H = h.shape
    Mp = wt.shape[2]
    BN = 512
    return pl.pallas_call(
        _dlogit_kernel,
        grid=(N // BN,),
        in_specs=[
            pl.BlockSpec((BN, H), lambda i: (i, 0)),
            pl.BlockSpec((2, H, Mp), lambda i: (0, 0, 0)),
            pl.BlockSpec((2, Mp), lambda i: (0, 0)),
        ],
        out_specs=pl.BlockSpec((BN, Mp), lambda i: (i, 0)),
        out_shape=jax.ShapeDtypeStruct((N, Mp), jnp.float32),
    )(h, wt, bt)


def _path_loss_sc(dmat, tgt, codes):
    N, Mp = dmat.shape
    V, D = codes.shape
    info = plsc.get_sparse_core_info()
    NC, NS, L = info.num_cores, info.num_subcores, info.num_lanes
    NW = NC * NS
    TPW = N // NW
    CH = 64

    @functools.partial(
        pl.kernel,
        mesh=plsc.VectorSubcoreMesh(core_axis_name="c", subcore_axis_name="s"),
        out_type=jax.ShapeDtypeStruct((NW, L), jnp.float32),
        compiler_params=pltpu.CompilerParams(needs_layout_passes=False),
        scratch_types=[
            pltpu.VMEM((TPW,), jnp.int32),
            pltpu.VMEM((CH, Mp), jnp.float32),
            pltpu.VMEM((V, D), jnp.int32),
            pltpu.VMEM((L,), jnp.float32),
        ],
    )
    def k(dmat_hbm, tgt_hbm, codes_hbm, out_hbm, tgt_v, d_v, c_v, o_v):
        wid = lax.axis_index("s") * NC + lax.axis_index("c")
        base = wid * TPW
        pltpu.sync_copy(tgt_hbm.at[pl.ds(base, TPW)], tgt_v)
        pltpu.sync_copy(codes_hbm, c_v)

        lanes = lax.iota(jnp.int32, L)
        one = jnp.float32(1.0)

        acc = jnp.zeros((L,), jnp.float32)
        for ci in range(TPW // CH):
            pltpu.sync_copy(dmat_hbm.at[pl.ds(base + ci * CH, CH)], d_v)

            def body_g(g, acc):
                tok = g * L + lanes
                v = tgt_v[pl.ds(ci * CH + g * L, L)]
                for j in range(D):
                    jv = jnp.full((L,), j, jnp.int32)
                    c = plsc.load_gather(c_v, [v, jv])
                    maskf = (c & 1).astype(jnp.float32)
                    sig = ((c >> 1) & 1).astype(jnp.float32) * 2.0 - 1.0
                    m = c >> 2
                    d = plsc.load_gather(d_v, [tok, m])
                    x = sig * d
                    p = one / (one + jnp.exp(-x))
                    u = jnp.exp(p) + jnp.exp(one - p)
                    y = jnp.full((L,), 1.2528, jnp.float32)
                    y = y + u * jnp.exp(-y) - one
                    y = y + u * jnp.exp(-y) - one
                    y = y + u * jnp.exp(-y) - one
                    acc = acc - maskf * (p - y)
                return acc

            acc = lax.fori_loop(0, CH // L, body_g, acc)
        o_v[...] = acc
        pltpu.sync_copy(o_v, out_hbm.at[wid])

    return k(dmat, tgt, codes)


def kernel(hidden, target, W, b, path_nodes, path_bits, path_mask):
    H = hidden.shape[-1]
    h = hidden.reshape(-1, H)
    t = target.reshape(-1).astype(jnp.int32)
    M = W.shape[0]
    Mp = (M + 7) // 8 * 8

    wp = jnp.pad(W, ((0, Mp - M), (0, 0), (0, 0)))
    bp = jnp.pad(b, ((0, Mp - M), (0, 0)))
    wt = jnp.transpose(wp, (1, 2, 0))
    bt = jnp.transpose(bp, (1, 0))

    codes = (
        (path_nodes.astype(jnp.int32) << 2)
        | (path_bits.astype(jnp.int32) << 1)
        | path_mask.astype(jnp.int32)
    )

    dmat = _dlogit_matmul(h, wt, bt)
    partial = _path_loss_sc(dmat, t, codes)
    return jnp.sum(partial)
